# baseline (speedup 1.0000x reference)
import functools

import jax
import jax.numpy as jnp
from jax import lax
from jax.experimental import pallas as pl
from jax.experimental.pallas import tpu as pltpu
from jax.experimental.pallas import tpu_sc as plsc

SEQ_LEN = 8192
MODEL_DIM = 1024
NUM_CORES = 2
ROWS_PER_WORKER = SEQ_LEN // NUM_CORES
CHUNK_ROWS = 512
NUM_CHUNKS = ROWS_PER_WORKER // CHUNK_ROWS
NBUF = 3

_mesh = plsc.ScalarSubcoreMesh(axis_name="c", num_cores=NUM_CORES)


@functools.partial(
    pl.kernel,
    mesh=_mesh,
    out_type=jax.ShapeDtypeStruct((SEQ_LEN, MODEL_DIM), jnp.float32),
    scratch_types=(
        [pltpu.VMEM_SHARED((NBUF, CHUNK_ROWS, MODEL_DIM), jnp.float32)]
        + [pltpu.SemaphoreType.DMA] * (2 * NBUF)
    ),
)
def _identity_gather(emb_hbm, out_hbm, buf, *sems):
    in_sems = sems[:NBUF]
    out_sems = sems[NBUF:]
    wid = lax.axis_index("c")
    base = wid * ROWS_PER_WORKER

    def chunk_slice(i):
        return pl.ds(base + i * CHUNK_ROWS, CHUNK_ROWS)

    for i in range(NBUF - 1):
        pltpu.async_copy(emb_hbm.at[chunk_slice(i)], buf.at[i], in_sems[i])
    for i in range(NUM_CHUNKS):
        cur = i % NBUF
        j = i + NBUF - 1
        if j < NUM_CHUNKS:
            b = j % NBUF
            if j >= NBUF:
                pltpu.make_async_copy(
                    buf.at[b], out_hbm.at[chunk_slice(j - NBUF)], out_sems[b]
                ).wait()
            pltpu.async_copy(emb_hbm.at[chunk_slice(j)], buf.at[b], in_sems[b])
        pltpu.make_async_copy(
            emb_hbm.at[chunk_slice(i)], buf.at[cur], in_sems[cur]
        ).wait()
        pltpu.async_copy(buf.at[cur], out_hbm.at[chunk_slice(i)], out_sems[cur])
    for i in range(max(0, NUM_CHUNKS - NBUF), NUM_CHUNKS):
        cur = i % NBUF
        pltpu.make_async_copy(
            buf.at[cur], out_hbm.at[chunk_slice(i)], out_sems[cur]
        ).wait()


def kernel(x, emb):
    del x
    return _identity_gather(emb)


# speedup vs baseline: 1.0784x; 1.0784x over previous
import functools

import jax
import jax.numpy as jnp
from jax import lax
from jax.experimental import pallas as pl
from jax.experimental.pallas import tpu as pltpu
from jax.experimental.pallas import tpu_sc as plsc

SEQ_LEN = 8192
MODEL_DIM = 1024
NUM_CORES = 2
NUM_SUBCORES = 16
NUM_WORKERS = NUM_CORES * NUM_SUBCORES
ROWS_PER_WORKER = SEQ_LEN // NUM_WORKERS
CHUNK_ROWS = 32
NUM_CHUNKS = ROWS_PER_WORKER // CHUNK_ROWS
NBUF = 3

_mesh = plsc.VectorSubcoreMesh(core_axis_name="c", subcore_axis_name="s")


@functools.partial(
    pl.kernel,
    mesh=_mesh,
    out_type=jax.ShapeDtypeStruct((SEQ_LEN, MODEL_DIM), jnp.float32),
    scratch_types=(
        [pltpu.VMEM((NBUF, CHUNK_ROWS, MODEL_DIM), jnp.float32)]
        + [pltpu.SemaphoreType.DMA] * (2 * NBUF)
    ),
)
def _identity_gather(emb_hbm, out_hbm, buf, *sems):
    in_sems = sems[:NBUF]
    out_sems = sems[NBUF:]
    wid = lax.axis_index("s") * NUM_CORES + lax.axis_index("c")
    base = wid * ROWS_PER_WORKER

    def chunk_slice(i):
        return pl.ds(base + i * CHUNK_ROWS, CHUNK_ROWS)

    for i in range(NBUF - 1):
        pltpu.async_copy(emb_hbm.at[chunk_slice(i)], buf.at[i], in_sems[i])
    for i in range(NUM_CHUNKS):
        cur = i % NBUF
        j = i + NBUF - 1
        if j < NUM_CHUNKS:
            b = j % NBUF
            if j >= NBUF:
                pltpu.make_async_copy(
                    buf.at[b], out_hbm.at[chunk_slice(j - NBUF)], out_sems[b]
                ).wait()
            pltpu.async_copy(emb_hbm.at[chunk_slice(j)], buf.at[b], in_sems[b])
        pltpu.make_async_copy(
            emb_hbm.at[chunk_slice(i)], buf.at[cur], in_sems[cur]
        ).wait()
        pltpu.async_copy(buf.at[cur], out_hbm.at[chunk_slice(i)], out_sems[cur])
    for i in range(max(0, NUM_CHUNKS - NBUF), NUM_CHUNKS):
        cur = i % NBUF
        pltpu.make_async_copy(
            buf.at[cur], out_hbm.at[chunk_slice(i)], out_sems[cur]
        ).wait()


def kernel(x, emb):
    del x
    return _identity_gather(emb)


# speedup vs baseline: 1.0808x; 1.0022x over previous
import functools

import jax
import jax.numpy as jnp
from jax import lax
from jax.experimental import pallas as pl
from jax.experimental.pallas import tpu as pltpu
from jax.experimental.pallas import tpu_sc as plsc

SEQ_LEN = 8192
MODEL_DIM = 1024
NUM_CORES = 2
NUM_SUBCORES = 16
NUM_WORKERS = NUM_CORES * NUM_SUBCORES
ROWS_PER_WORKER = SEQ_LEN // NUM_WORKERS
CHUNK_ROWS = 32
NUM_CHUNKS = ROWS_PER_WORKER // CHUNK_ROWS
NBUF = 3

_mesh = plsc.VectorSubcoreMesh(core_axis_name="c", subcore_axis_name="s")


@functools.partial(
    pl.kernel,
    mesh=_mesh,
    out_type=jax.ShapeDtypeStruct((SEQ_LEN, MODEL_DIM), jnp.float32),
    scratch_types=(
        [pltpu.VMEM((NBUF, CHUNK_ROWS, MODEL_DIM), jnp.float32)]
        + [pltpu.SemaphoreType.DMA] * (2 * NBUF)
    ),
)
def _identity_gather(emb_hbm, out_hbm, buf, *sems):
    in_sems = sems[:NBUF]
    out_sems = sems[NBUF:]
    wid = lax.axis_index("c") * NUM_SUBCORES + lax.axis_index("s")
    base = wid * ROWS_PER_WORKER

    def chunk_slice(i):
        return pl.ds(base + i * CHUNK_ROWS, CHUNK_ROWS)

    for i in range(NBUF - 1):
        pltpu.async_copy(emb_hbm.at[chunk_slice(i)], buf.at[i], in_sems[i])
    for i in range(NUM_CHUNKS):
        cur = i % NBUF
        j = i + NBUF - 1
        if j < NUM_CHUNKS:
            b = j % NBUF
            if j >= NBUF:
                pltpu.make_async_copy(
                    buf.at[b], out_hbm.at[chunk_slice(j - NBUF)], out_sems[b]
                ).wait()
            pltpu.async_copy(emb_hbm.at[chunk_slice(j)], buf.at[b], in_sems[b])
        pltpu.make_async_copy(
            emb_hbm.at[chunk_slice(i)], buf.at[cur], in_sems[cur]
        ).wait()
        pltpu.async_copy(buf.at[cur], out_hbm.at[chunk_slice(i)], out_sems[cur])
    for i in range(max(0, NUM_CHUNKS - NBUF), NUM_CHUNKS):
        cur = i % NBUF
        pltpu.make_async_copy(
            buf.at[cur], out_hbm.at[chunk_slice(i)], out_sems[cur]
        ).wait()


def kernel(x, emb):
    del x
    return _identity_gather(emb)
